# baseline (speedup 1.0000x reference)
import jax
import jax.numpy as jnp
from jax.experimental import pallas as pl
from jax.experimental.pallas import tpu as pltpu

_TOPK = 5
_RADIUS = 2
_T = 2
_NEG = -1e30


def _leaky(x):
    return jnp.where(x >= 0, x, 0.01 * x)


def _sig(x):
    return 1.0 / (1.0 + jnp.exp(-x))


def _elu(x):
    return jnp.where(x > 0, x, jnp.exp(jnp.minimum(x, 0.0)) - 1.0)


def _dot(a, b):
    return jnp.dot(a, b, preferred_element_type=jnp.float32)


def _top5_onehots(sim):
    vals, ones = [], []
    for _ in range(_TOPK):
        m = jnp.max(sim, axis=1, keepdims=True)
        eq = sim == m
        vals.append(m)
        ones.append(jnp.where(eq, 1.0, 0.0))
        sim = jnp.where(eq, _NEG, sim)
    return vals, ones


def _softmax_k(logits):
    m = logits[0]
    for l in logits[1:]:
        m = jnp.maximum(m, l)
    es = [jnp.exp(l - m) for l in logits]
    s = es[0]
    for e in es[1:]:
        s = s + e
    return [e / s for e in es]


def kernel(input, params):
    p = params
    B, N, RAW = input.shape
    F = p[0].shape[0]

    tr = lambda w: w.T
    mats = jnp.stack(
        [tr(p[40]), tr(p[0]), tr(p[2][:, :F]), tr(p[16]), tr(p[18]),
         tr(p[26]), tr(p[34])]
        + [tr(p[4][i * F:(i + 1) * F]) for i in range(3)]
        + [tr(p[5][i * F:(i + 1) * F]) for i in range(3)]
        + [tr(p[8][i * F:(i + 1) * F]) for i in range(3)]
        + [tr(p[9][i * F:(i + 1) * F]) for i in range(3)]
        + [tr(p[20][i * F:(i + 1) * F]) for i in range(3)]
        + [tr(p[21][i * F:(i + 1) * F]) for i in range(3)]
        + [tr(p[28][i * F:(i + 1) * F]) for i in range(3)]
        + [tr(p[29][i * F:(i + 1) * F]) for i in range(3)]
    )

    vecs = jnp.stack(
        [p[39], p[41], p[42], p[1], p[2][:, F], p[3],
         p[12][0, :F], p[12][0, F:], p[17],
         p[14][0, :F], p[14][0, F:], p[19]]
        + [p[6][i * F:(i + 1) * F] for i in range(3)]
        + [p[7][i * F:(i + 1) * F] for i in range(3)]
        + [p[10][i * F:(i + 1) * F] for i in range(3)]
        + [p[11][i * F:(i + 1) * F] for i in range(3)]
        + [p[22][i * F:(i + 1) * F] for i in range(3)]
        + [p[23][i * F:(i + 1) * F] for i in range(3)]
        + [p[30][i * F:(i + 1) * F] for i in range(3)]
        + [p[31][i * F:(i + 1) * F] for i in range(3)]
        + [p[24][0, :F], p[24][0, F:], p[27],
           p[32][0, :F], p[32][0, F:], p[35]]
    )[:, None, :]

    scal = jnp.stack([p[13][0], p[15][0], p[25][0], p[33][0]]).reshape(4, 1, 1)
    W38t = p[38].T
    W36t = p[36].T
    b37 = p[37][None]
    OUT = W36t.shape[1]

    def body(x_ref, w38_ref, mats_ref, vecs_ref, scal_ref, w36_ref, b37_ref,
             out_ref, xbuf, sems):
        V = lambda i: vecs_ref[i]
        M = lambda i: mats_ref[i]
        S = lambda i: scal_ref[i]

        def gru(x, h, mi, vi):
            r = _sig(_dot(x, M(mi)) + V(vi) + _dot(h, M(mi + 3)) + V(vi + 3))
            z = _sig(_dot(x, M(mi + 1)) + V(vi + 1)
                     + _dot(h, M(mi + 4)) + V(vi + 4))
            nn = jnp.tanh(_dot(x, M(mi + 2)) + V(vi + 2)
                          + r * (_dot(h, M(mi + 5)) + V(vi + 5)))
            return (1.0 - z) * nn + z * h

        b = pl.program_id(0)

        def copy_in(i, slot):
            pltpu.make_async_copy(x_ref.at[i], xbuf.at[slot],
                                  sems.at[slot]).start()

        @pl.when(b == 0)
        def _():
            copy_in(0, 0)

        @pl.when(b < B - 1)
        def _():
            copy_in(b + 1, (b + 1) % 2)

        slot = b % 2
        pltpu.make_async_copy(x_ref.at[b], xbuf.at[slot], sems.at[slot]).wait()
        x = xbuf[slot].astype(jnp.bfloat16)
        node_list = _leaky(
            _dot(x, w38_ref[...].astype(jnp.bfloat16)) + V(0))

        h = _leaky(_dot(node_list, M(0)) + V(1)) * V(2)
        sim = _dot(h, h.T)
        vals, ones = _top5_onehots(sim)

        node_feature = _leaky(_dot(node_list, M(1)) + V(3))
        P = _dot(node_list, M(2))
        nfs = [_leaky(_dot(o, P) + v * V(4) + V(5))
               for o, v in zip(ones, vals)]

        u = jnp.sum(node_feature * V(6), axis=1, keepdims=True)
        logits = [_leaky(u + jnp.sum(nf * V(7), axis=1, keepdims=True) + S(0))
                  for nf in nfs]
        aws = _softmax_k(logits)
        Ssum = None
        for a, nf in zip(aws, nfs):
            Ssum = a * nf if Ssum is None else Ssum + a * nf
        ctx = _elu(_dot(Ssum, M(3)) + V(8))
        nfr = gru(ctx, node_feature, 7, 12)
        act = jnp.maximum(nfr, 0.0)

        for _ in range(_RADIUS - 1):
            h2 = _leaky(_dot(act, M(0)) + V(1)) * V(2)
            sim2 = _dot(h2, h2.T)
            _, ones2 = _top5_onehots(sim2)
            q = jnp.sum(act * V(10), axis=1, keepdims=True)
            w2s = [_dot(o, q) for o in ones2]
            u2 = jnp.sum(act * V(9), axis=1, keepdims=True)
            logits2 = [_leaky(u2 + w2 + S(1)) for w2 in w2s]
            aws2 = _softmax_k(logits2)
            Wc = None
            for a, o in zip(aws2, ones2):
                Wc = a * o if Wc is None else Wc + a * o
            ctx2 = _elu(_dot(_dot(Wc, act), M(4)) + V(11))
            nfr = gru(ctx2, nfr, 13, 18)
            act = jnp.maximum(nfr, 0.0)

        gf = jnp.sum(act, axis=0, keepdims=True)
        agf = jnp.maximum(gf, 0.0)
        gf2, agf2 = gf, agf

        wnA = jnp.sum(act * V(37), axis=1, keepdims=True)
        wnB = jnp.sum(act * V(40), axis=1, keepdims=True)

        for _ in range(_T):
            uu = jnp.sum(agf * V(36), axis=1, keepdims=True)
            lg = _leaky(wnA + uu + S(2))
            mm = jnp.max(lg, axis=0, keepdims=True)
            e = jnp.exp(lg - mm)
            gw = e / jnp.sum(e, axis=0, keepdims=True)
            gvec = jnp.sum(gw * act, axis=0, keepdims=True)
            gctx = _elu(_dot(gvec, M(5)) + V(38))
            gf = gru(gctx, gf, 19, 24)
            agf = jnp.maximum(gf, 0.0)

        for _ in range(_T):
            uu = jnp.sum(agf2 * V(39), axis=1, keepdims=True)
            lg = _leaky(wnB + uu + S(3))
            mm = jnp.max(lg, axis=0, keepdims=True)
            e = jnp.exp(lg - mm)
            gw = e / jnp.sum(e, axis=0, keepdims=True)
            gvec = jnp.sum(gw * act, axis=0, keepdims=True)
            gctx = _elu(_dot(gvec, M(6)) + V(41))
            gf2 = gru(gctx, gf2, 25, 30)
            agf2 = jnp.maximum(gf2, 0.0)

        out_ref[0] = _dot(gf + gf2, w36_ref[...]) + b37_ref[...]

    out3 = pl.pallas_call(
        body,
        grid=(B,),
        in_specs=[
            pl.BlockSpec(memory_space=pl.ANY),
            pl.BlockSpec((RAW, F), lambda b: (0, 0)),
            pl.BlockSpec(mats.shape, lambda b: (0, 0, 0)),
            pl.BlockSpec(vecs.shape, lambda b: (0, 0, 0)),
            pl.BlockSpec(scal.shape, lambda b: (0, 0, 0)),
            pl.BlockSpec((F, OUT), lambda b: (0, 0)),
            pl.BlockSpec((1, OUT), lambda b: (0, 0)),
        ],
        out_specs=pl.BlockSpec((1, 1, OUT), lambda b: (b, 0, 0)),
        out_shape=jax.ShapeDtypeStruct((B, 1, OUT), jnp.float32),
        compiler_params=pltpu.CompilerParams(
            vmem_limit_bytes=128 * 1024 * 1024),
        scratch_shapes=[pltpu.VMEM((2, N, RAW), jnp.float32),
                        pltpu.SemaphoreType.DMA((2,))],
    )(input, W38t, mats, vecs, scal, W36t, b37)
    return out3.reshape(B, OUT)


# speedup vs baseline: 1.0212x; 1.0212x over previous
import jax
import jax.numpy as jnp
from jax.experimental import pallas as pl
from jax.experimental.pallas import tpu as pltpu

_TOPK = 5
_RADIUS = 2
_T = 2
_NEG = -1e30


def _leaky(x):
    return jnp.maximum(x, 0.01 * x)


def _sig(x):
    return jax.nn.sigmoid(x)


def _elu(x):
    return jnp.maximum(x, jnp.exp(jnp.minimum(x, 0.0)) - 1.0)


def _dot(a, b):
    return jnp.dot(a, b, preferred_element_type=jnp.float32)


def _top5_onehots(sim):
    vals, ones = [], []
    for _ in range(_TOPK):
        m = jnp.max(sim, axis=1, keepdims=True)
        eq = sim == m
        vals.append(m)
        ones.append(eq.astype(jnp.float32))
        sim = jnp.where(eq, _NEG, sim)
    return vals, ones


def _softmax_k(logits):
    m = logits[0]
    for l in logits[1:]:
        m = jnp.maximum(m, l)
    es = [jnp.exp(l - m) for l in logits]
    s = es[0]
    for e in es[1:]:
        s = s + e
    return [e / s for e in es]


def kernel(input, params):
    p = params
    B, N, RAW = input.shape
    F = p[0].shape[0]

    tr = lambda w: w.T
    mats = jnp.stack(
        [tr(p[40]), tr(p[0]), tr(p[2][:, :F]), tr(p[16]), tr(p[18]),
         tr(p[26]), tr(p[34])]
        + [tr(p[4][i * F:(i + 1) * F]) for i in range(3)]
        + [tr(p[5][i * F:(i + 1) * F]) for i in range(3)]
        + [tr(p[8][i * F:(i + 1) * F]) for i in range(3)]
        + [tr(p[9][i * F:(i + 1) * F]) for i in range(3)]
        + [tr(p[20][i * F:(i + 1) * F]) for i in range(3)]
        + [tr(p[21][i * F:(i + 1) * F]) for i in range(3)]
        + [tr(p[28][i * F:(i + 1) * F]) for i in range(3)]
        + [tr(p[29][i * F:(i + 1) * F]) for i in range(3)]
    )

    vecs = jnp.stack(
        [p[39], p[41], p[42], p[1], p[2][:, F], p[3],
         p[12][0, :F], p[12][0, F:], p[17],
         p[14][0, :F], p[14][0, F:], p[19]]
        + [p[6][i * F:(i + 1) * F] for i in range(3)]
        + [p[7][i * F:(i + 1) * F] for i in range(3)]
        + [p[10][i * F:(i + 1) * F] for i in range(3)]
        + [p[11][i * F:(i + 1) * F] for i in range(3)]
        + [p[22][i * F:(i + 1) * F] for i in range(3)]
        + [p[23][i * F:(i + 1) * F] for i in range(3)]
        + [p[30][i * F:(i + 1) * F] for i in range(3)]
        + [p[31][i * F:(i + 1) * F] for i in range(3)]
        + [p[24][0, :F], p[24][0, F:], p[27],
           p[32][0, :F], p[32][0, F:], p[35]]
    )[:, None, :]

    scal = jnp.stack([p[13][0], p[15][0], p[25][0], p[33][0]]).reshape(4, 1, 1)
    W38t = p[38].T
    W36t = p[36].T
    b37 = p[37][None]
    OUT = W36t.shape[1]

    def body(x_ref, w38_ref, mats_ref, vecs_ref, scal_ref, w36_ref, b37_ref,
             out_ref):
        V = lambda i: vecs_ref[i]
        M = lambda i: mats_ref[i]
        S = lambda i: scal_ref[i]

        def gru(x, h, mi, vi):
            r = _sig(_dot(x, M(mi)) + V(vi) + _dot(h, M(mi + 3)) + V(vi + 3))
            z = _sig(_dot(x, M(mi + 1)) + V(vi + 1)
                     + _dot(h, M(mi + 4)) + V(vi + 4))
            nn = jnp.tanh(_dot(x, M(mi + 2)) + V(vi + 2)
                          + r * (_dot(h, M(mi + 5)) + V(vi + 5)))
            return (1.0 - z) * nn + z * h

        x = x_ref[0].astype(jnp.bfloat16)
        node_list = _leaky(
            _dot(x, w38_ref[...].astype(jnp.bfloat16)) + V(0))

        h = _leaky(_dot(node_list, M(0)) + V(1)) * V(2)
        sim = _dot(h, h.T)
        vals, ones = _top5_onehots(sim)

        node_feature = _leaky(_dot(node_list, M(1)) + V(3))
        P = _dot(node_list, M(2))
        nfs = [_leaky(_dot(o, P) + v * V(4) + V(5))
               for o, v in zip(ones, vals)]

        u = jnp.sum(node_feature * V(6), axis=1, keepdims=True)
        logits = [_leaky(u + jnp.sum(nf * V(7), axis=1, keepdims=True) + S(0))
                  for nf in nfs]
        aws = _softmax_k(logits)
        Ssum = None
        for a, nf in zip(aws, nfs):
            Ssum = a * nf if Ssum is None else Ssum + a * nf
        ctx = _elu(_dot(Ssum, M(3)) + V(8))
        nfr = gru(ctx, node_feature, 7, 12)
        act = jnp.maximum(nfr, 0.0)

        for _ in range(_RADIUS - 1):
            h2 = _leaky(_dot(act, M(0)) + V(1)) * V(2)
            sim2 = _dot(h2, h2.T)
            _, ones2 = _top5_onehots(sim2)
            q = jnp.sum(act * V(10), axis=1, keepdims=True)
            w2s = [_dot(o, q) for o in ones2]
            u2 = jnp.sum(act * V(9), axis=1, keepdims=True)
            logits2 = [_leaky(u2 + w2 + S(1)) for w2 in w2s]
            aws2 = _softmax_k(logits2)
            Wc = None
            for a, o in zip(aws2, ones2):
                Wc = a * o if Wc is None else Wc + a * o
            ctx2 = _elu(_dot(_dot(Wc, act), M(4)) + V(11))
            nfr = gru(ctx2, nfr, 13, 18)
            act = jnp.maximum(nfr, 0.0)

        gf = jnp.sum(act, axis=0, keepdims=True)
        agf = jnp.maximum(gf, 0.0)
        gf2, agf2 = gf, agf

        wnA = jnp.sum(act * V(37), axis=1, keepdims=True)
        wnB = jnp.sum(act * V(40), axis=1, keepdims=True)

        for _ in range(_T):
            uu = jnp.sum(agf * V(36), axis=1, keepdims=True)
            lg = _leaky(wnA + uu + S(2))
            mm = jnp.max(lg, axis=0, keepdims=True)
            e = jnp.exp(lg - mm)
            gw = e / jnp.sum(e, axis=0, keepdims=True)
            gvec = jnp.sum(gw * act, axis=0, keepdims=True)
            gctx = _elu(_dot(gvec, M(5)) + V(38))
            gf = gru(gctx, gf, 19, 24)
            agf = jnp.maximum(gf, 0.0)

        for _ in range(_T):
            uu = jnp.sum(agf2 * V(39), axis=1, keepdims=True)
            lg = _leaky(wnB + uu + S(3))
            mm = jnp.max(lg, axis=0, keepdims=True)
            e = jnp.exp(lg - mm)
            gw = e / jnp.sum(e, axis=0, keepdims=True)
            gvec = jnp.sum(gw * act, axis=0, keepdims=True)
            gctx = _elu(_dot(gvec, M(6)) + V(41))
            gf2 = gru(gctx, gf2, 25, 30)
            agf2 = jnp.maximum(gf2, 0.0)

        out_ref[0] = _dot(gf + gf2, w36_ref[...]) + b37_ref[...]

    out3 = pl.pallas_call(
        body,
        grid=(B,),
        in_specs=[
            pl.BlockSpec((1, N, RAW), lambda b: (b, 0, 0)),
            pl.BlockSpec((RAW, F), lambda b: (0, 0)),
            pl.BlockSpec(mats.shape, lambda b: (0, 0, 0)),
            pl.BlockSpec(vecs.shape, lambda b: (0, 0, 0)),
            pl.BlockSpec(scal.shape, lambda b: (0, 0, 0)),
            pl.BlockSpec((F, OUT), lambda b: (0, 0)),
            pl.BlockSpec((1, OUT), lambda b: (0, 0)),
        ],
        out_specs=pl.BlockSpec((1, 1, OUT), lambda b: (b, 0, 0)),
        out_shape=jax.ShapeDtypeStruct((B, 1, OUT), jnp.float32),
        compiler_params=pltpu.CompilerParams(
            vmem_limit_bytes=128 * 1024 * 1024),
    )(input, W38t, mats, vecs, scal, W36t, b37)
    return out3.reshape(B, OUT)
